# baseline (speedup 1.0000x reference)
import functools

import jax
import jax.numpy as jnp
from jax import lax
from jax.experimental import pallas as pl
from jax.experimental.pallas import tpu as pltpu
from jax.experimental.pallas import tpu_sc as plsc

L = 16
NC = 2
NS = 16
NW = NC * NS
B = 16384
K = 16
BPW = B // NW
GROUPS = BPW // L
CH = 4
CPW = BPW // CH
CG = CPW // L
ROWS_PER_LINE = 8


def _body(uid_hbm, iid_hbm, uemb_hbm, ubias_hbm, iemb_hbm, ibias_hbm,
          bias_hbm, out_hbm,
          uidx_v, iidx_v, ulin_v, ilin_v, ulines_v, ilines_v,
          ub_v, ib_v, bias_v, out_v, bsem, lsem):
    wid = lax.axis_index("s") * NC + lax.axis_index("c")
    base = wid * BPW

    pltpu.sync_copy(uid_hbm.at[pl.ds(base, BPW)], uidx_v)
    pltpu.sync_copy(iid_hbm.at[pl.ds(base, BPW)], iidx_v)

    bcp1 = pltpu.async_copy(ubias_hbm.at[uidx_v], ub_v, bsem)
    bcp2 = pltpu.async_copy(ibias_hbm.at[iidx_v], ib_v, bsem)
    pltpu.sync_copy(bias_hbm, bias_v)

    def mkidx(g, carry):
        rb = g * L
        ulin_v[pl.ds(rb, L)] = lax.shift_right_logical(
            uidx_v[pl.ds(rb, L)], 3)
        ilin_v[pl.ds(rb, L)] = lax.shift_right_logical(
            iidx_v[pl.ds(rb, L)], 3)
        return carry

    lax.fori_loop(0, GROUPS, mkidx, 0)

    bcp1.wait()
    bcp2.wait()
    bvec = bias_v[...]
    lane = lax.iota(jnp.int32, L)

    for c in range(CH):
        cb = c * CPW
        lcp1 = pltpu.async_copy(
            uemb_hbm.at[ulin_v.at[pl.ds(cb, CPW)]], ulines_v, lsem)
        lcp2 = pltpu.async_copy(
            iemb_hbm.at[ilin_v.at[pl.ds(cb, CPW)]], ilines_v, lsem)
        lcp1.wait()
        lcp2.wait()
        for g in range(CG):
            rb = cb + g * L
            rows = lane + g * L
            sub_u = (uidx_v[pl.ds(rb, L)] & 7) << 4
            sub_i = (iidx_v[pl.ds(rb, L)] & 7) << 4
            acc = ub_v[pl.ds(rb, L)] + ib_v[pl.ds(rb, L)] + bvec
            for k in range(K):
                acc = acc + (plsc.load_gather(ulines_v, [rows, sub_u + k]) *
                             plsc.load_gather(ilines_v, [rows, sub_i + k]))
            out_v[pl.ds(rb, L)] = acc

    pltpu.sync_copy(out_v, out_hbm.at[pl.ds(base, BPW)])


_mesh = plsc.VectorSubcoreMesh(core_axis_name="c", subcore_axis_name="s")

_sc_call = functools.partial(
    pl.kernel,
    out_type=jax.ShapeDtypeStruct((B,), jnp.float32),
    mesh=_mesh,
    compiler_params=pltpu.CompilerParams(needs_layout_passes=False,
                                         use_tc_tiling_on_sc=False),
    scratch_types=[
        pltpu.VMEM((BPW,), jnp.int32),
        pltpu.VMEM((BPW,), jnp.int32),
        pltpu.VMEM((BPW,), jnp.int32),
        pltpu.VMEM((BPW,), jnp.int32),
        pltpu.VMEM((CPW, 128), jnp.float32),
        pltpu.VMEM((CPW, 128), jnp.float32),
        pltpu.VMEM((BPW,), jnp.float32),
        pltpu.VMEM((BPW,), jnp.float32),
        pltpu.VMEM((L,), jnp.float32),
        pltpu.VMEM((BPW,), jnp.float32),
        pltpu.SemaphoreType.DMA,
        pltpu.SemaphoreType.DMA,
    ],
)(_body)


@jax.jit
def kernel(user_id, item_id, user_emb, user_bias, item_emb, item_bias, bias):
    bias16 = jnp.broadcast_to(bias.astype(jnp.float32), (L,))
    uemb_lines = user_emb.reshape(-1, ROWS_PER_LINE * K)
    iemb_lines = item_emb.reshape(-1, ROWS_PER_LINE * K)
    return _sc_call(user_id.astype(jnp.int32), item_id.astype(jnp.int32),
                    uemb_lines, user_bias, iemb_lines, item_bias, bias16)


# speedup vs baseline: 4.1690x; 4.1690x over previous
import functools

import jax
import jax.numpy as jnp
from jax import lax
from jax.experimental import pallas as pl
from jax.experimental.pallas import tpu as pltpu
from jax.experimental.pallas import tpu_sc as plsc

L = 16
NC = 2
NS = 16
NW = NC * NS
B = 16384
K = 16
BPW = B // NW
D = 8
GENS = BPW // D
BLK = 128

_mesh = plsc.VectorSubcoreMesh(core_axis_name="c", subcore_axis_name="s")


def _bias_body(uid_hbm, iid_hbm, ubias_hbm, ibias_hbm, bias_hbm, out_hbm,
               uidx_v, iidx_v, ub_v, ib_v, bias_v, out_v, bsem):
    wid = lax.axis_index("s") * NC + lax.axis_index("c")
    base = wid * BPW
    pltpu.sync_copy(uid_hbm.at[pl.ds(base, BPW)], uidx_v)
    pltpu.sync_copy(iid_hbm.at[pl.ds(base, BPW)], iidx_v)
    bcp1 = pltpu.async_copy(ubias_hbm.at[uidx_v], ub_v, bsem)
    bcp2 = pltpu.async_copy(ibias_hbm.at[iidx_v], ib_v, bsem)
    pltpu.sync_copy(bias_hbm, bias_v)
    bcp1.wait()
    bcp2.wait()
    bvec = bias_v[...]

    def group(g, carry):
        rb = g * L
        out_v[pl.ds(rb, L)] = ub_v[pl.ds(rb, L)] + ib_v[pl.ds(rb, L)] + bvec
        return carry

    lax.fori_loop(0, BPW // L, group, 0)
    pltpu.sync_copy(out_v, out_hbm.at[pl.ds(base, BPW)])


_bias_call = functools.partial(
    pl.kernel,
    out_type=jax.ShapeDtypeStruct((B,), jnp.float32),
    mesh=_mesh,
    compiler_params=pltpu.CompilerParams(needs_layout_passes=False,
                                         use_tc_tiling_on_sc=False),
    scratch_types=[
        pltpu.VMEM((BPW,), jnp.int32),
        pltpu.VMEM((BPW,), jnp.int32),
        pltpu.VMEM((BPW,), jnp.float32),
        pltpu.VMEM((BPW,), jnp.float32),
        pltpu.VMEM((L,), jnp.float32),
        pltpu.VMEM((BPW,), jnp.float32),
        pltpu.SemaphoreType.DMA,
    ],
)(_bias_body)


def _dot_body(uid_hbm, iid_hbm, uembt_hbm, iembt_hbm, pb_hbm, out_hbm,
              uidx_v, iidx_v, uring_v, iring_v, pb_v, out_v, *slot_sems):
    usems = slot_sems[:D]
    isems = slot_sems[D:]
    wid = lax.axis_index("s") * NC + lax.axis_index("c")
    base = wid * BPW

    pltpu.sync_copy(uid_hbm.at[pl.ds(base, BPW)], uidx_v)
    pltpu.sync_copy(iid_hbm.at[pl.ds(base, BPW)], iidx_v)
    pltpu.sync_copy(pb_hbm.at[pl.ds(base, BPW)], pb_v)

    def fire(gen, d):
        uids = uidx_v[pl.ds(gen * D, L)]
        iids = iidx_v[pl.ds(gen * D, L)]
        u_off = pl.multiple_of((uids[d] >> 7) << 7, BLK)
        i_off = pl.multiple_of((iids[d] >> 7) << 7, BLK)
        pltpu.async_copy(uembt_hbm.at[:, pl.ds(u_off, BLK)],
                         uring_v.at[d], usems[d])
        pltpu.async_copy(iembt_hbm.at[:, pl.ds(i_off, BLK)],
                         iring_v.at[d], isems[d])

    for d in range(D):
        fire(0, d)

    lane = lax.iota(jnp.int32, L)
    zeros = jnp.zeros((L,), jnp.int32)

    def gen_step(g2, carry):
        rb = g2 * L
        uids = uidx_v[pl.ds(rb, L)]
        iids = iidx_v[pl.ds(rb, L)]
        acc = pb_v[pl.ds(rb, L)]
        for h in range(L // D):
            g = g2 * (L // D) + h
            for d in range(D):
                pltpu.make_async_copy(uembt_hbm.at[:, pl.ds(0, BLK)],
                                      uring_v.at[d], usems[d]).wait()
                pltpu.make_async_copy(iembt_hbm.at[:, pl.ds(0, BLK)],
                                      iring_v.at[d], isems[d]).wait()
                lane_idx = h * D + d
                ucol = (uids[lane_idx] & 127) + zeros
                icol = (iids[lane_idx] & 127) + zeros
                uvec = plsc.load_gather(uring_v.at[d], [lane, ucol])
                ivec = plsc.load_gather(iring_v.at[d], [lane, icol])
                s = jnp.sum(uvec * ivec)
                acc = jnp.where(lane == lane_idx, acc + s, acc)

            @pl.when(g < GENS - 1)
            def _refire():
                for d in range(D):
                    fire(g + 1, d)

        out_v[pl.ds(rb, L)] = acc
        return carry

    lax.fori_loop(0, BPW // L, gen_step, 0)

    pltpu.sync_copy(out_v, out_hbm.at[pl.ds(base, BPW)])


_dot_call = functools.partial(
    pl.kernel,
    out_type=jax.ShapeDtypeStruct((B,), jnp.float32),
    mesh=_mesh,
    compiler_params=pltpu.CompilerParams(needs_layout_passes=False),
    scratch_types=[
        pltpu.VMEM((BPW,), jnp.int32),
        pltpu.VMEM((BPW,), jnp.int32),
        pltpu.VMEM((D, K, BLK), jnp.float32),
        pltpu.VMEM((D, K, BLK), jnp.float32),
        pltpu.VMEM((BPW,), jnp.float32),
        pltpu.VMEM((BPW,), jnp.float32),
    ] + [pltpu.SemaphoreType.DMA] * (2 * D),
)(_dot_body)


@jax.jit
def kernel(user_id, item_id, user_emb, user_bias, item_emb, item_bias, bias):
    uid = user_id.astype(jnp.int32)
    iid = item_id.astype(jnp.int32)
    bias16 = jnp.broadcast_to(bias.astype(jnp.float32), (L,))
    pb = _bias_call(uid, iid, user_bias, item_bias, bias16)
    return _dot_call(uid, iid, user_emb.T, item_emb.T, pb)


# speedup vs baseline: 4.3850x; 1.0518x over previous
import functools

import jax
import jax.numpy as jnp
from jax import lax
from jax.experimental import pallas as pl
from jax.experimental.pallas import tpu as pltpu
from jax.experimental.pallas import tpu_sc as plsc

L = 16
NC = 2
NS = 16
NW = NC * NS
B = 16384
K = 16
BPW = B // NW
D = 8
GENS = BPW // D
BLK = 128
W = BLK

_mesh = plsc.VectorSubcoreMesh(core_axis_name="c", subcore_axis_name="s")


def _bias_body(uid_hbm, iid_hbm, ubias_hbm, ibias_hbm, bias_hbm, out_hbm,
               uidx_v, iidx_v, ub_v, ib_v, bias_v, out_v, bsem):
    wid = lax.axis_index("s") * NC + lax.axis_index("c")
    base = wid * BPW
    pltpu.sync_copy(uid_hbm.at[pl.ds(base, BPW)], uidx_v)
    pltpu.sync_copy(iid_hbm.at[pl.ds(base, BPW)], iidx_v)
    bcp1 = pltpu.async_copy(ubias_hbm.at[uidx_v], ub_v, bsem)
    bcp2 = pltpu.async_copy(ibias_hbm.at[iidx_v], ib_v, bsem)
    pltpu.sync_copy(bias_hbm, bias_v)
    bcp1.wait()
    bcp2.wait()
    bvec = bias_v[...]

    def group(g, carry):
        rb = g * L
        out_v[pl.ds(rb, L)] = ub_v[pl.ds(rb, L)] + ib_v[pl.ds(rb, L)] + bvec
        return carry

    lax.fori_loop(0, BPW // L, group, 0)
    pltpu.sync_copy(out_v, out_hbm.at[pl.ds(base, BPW)])


_bias_call = functools.partial(
    pl.kernel,
    out_type=jax.ShapeDtypeStruct((B,), jnp.float32),
    mesh=_mesh,
    compiler_params=pltpu.CompilerParams(needs_layout_passes=False,
                                         use_tc_tiling_on_sc=False),
    scratch_types=[
        pltpu.VMEM((BPW,), jnp.int32),
        pltpu.VMEM((BPW,), jnp.int32),
        pltpu.VMEM((BPW,), jnp.float32),
        pltpu.VMEM((BPW,), jnp.float32),
        pltpu.VMEM((L,), jnp.float32),
        pltpu.VMEM((BPW,), jnp.float32),
        pltpu.SemaphoreType.DMA,
    ],
)(_bias_body)


def _dot_body(uid_hbm, iid_hbm, uembt_hbm, iembt_hbm, pb_hbm, out_hbm,
              uidx_v, iidx_v, uring_v, iring_v, pb_v, out_v, *slot_sems):
    usems = slot_sems[:D]
    isems = slot_sems[D:]
    wid = lax.axis_index("s") * NC + lax.axis_index("c")
    base = wid * BPW

    pltpu.sync_copy(uid_hbm.at[pl.ds(base, BPW)], uidx_v.at[pl.ds(0, BPW)])
    pltpu.sync_copy(iid_hbm.at[pl.ds(base, BPW)], iidx_v.at[pl.ds(0, BPW)])
    pltpu.sync_copy(pb_hbm.at[pl.ds(base, BPW)], pb_v)

    def fire_gen(gen):
        uids = uidx_v[pl.ds(gen * D, L)]
        iids = iidx_v[pl.ds(gen * D, L)]
        for d in range(D):
            u_off = pl.multiple_of((uids[d] >> 7) << 7, BLK)
            i_off = pl.multiple_of((iids[d] >> 7) << 7, BLK)
            pltpu.async_copy(uembt_hbm.at[:, pl.ds(u_off, W)],
                             uring_v.at[d], usems[d])
            pltpu.async_copy(iembt_hbm.at[:, pl.ds(i_off, W)],
                             iring_v.at[d], isems[d])

    fire_gen(0)

    lane = lax.iota(jnp.int32, L)
    zeros = jnp.zeros((L,), jnp.int32)

    def gen_step(g2, carry):
        rb = g2 * L
        uids = uidx_v[pl.ds(rb, L)]
        iids = iidx_v[pl.ds(rb, L)]
        acc = pb_v[pl.ds(rb, L)]
        for h in range(L // D):
            g = g2 * (L // D) + h
            for d in range(D):
                pltpu.make_async_copy(uembt_hbm.at[:, pl.ds(0, W)],
                                      uring_v.at[d], usems[d]).wait()
                pltpu.make_async_copy(iembt_hbm.at[:, pl.ds(0, W)],
                                      iring_v.at[d], isems[d]).wait()
                lane_idx = h * D + d
                ucol = (uids[lane_idx] & (W - 1)) + zeros
                icol = (iids[lane_idx] & (W - 1)) + zeros
                uvec = plsc.load_gather(uring_v.at[d], [lane, ucol])
                ivec = plsc.load_gather(iring_v.at[d], [lane, icol])
                s = jnp.sum(uvec * ivec)
                acc = jnp.where(lane == lane_idx, acc + s, acc)

            @pl.when(g < GENS - 1)
            def _refire():
                fire_gen(g + 1)

        out_v[pl.ds(rb, L)] = acc
        return carry

    lax.fori_loop(0, BPW // L, gen_step, 0)

    pltpu.sync_copy(out_v, out_hbm.at[pl.ds(base, BPW)])


_dot_call = functools.partial(
    pl.kernel,
    out_type=jax.ShapeDtypeStruct((B,), jnp.float32),
    mesh=_mesh,
    compiler_params=pltpu.CompilerParams(needs_layout_passes=False),
    scratch_types=[
        pltpu.VMEM((BPW + L,), jnp.int32),
        pltpu.VMEM((BPW + L,), jnp.int32),
        pltpu.VMEM((D, K, W), jnp.float32),
        pltpu.VMEM((D, K, W), jnp.float32),
        pltpu.VMEM((BPW,), jnp.float32),
        pltpu.VMEM((BPW,), jnp.float32),
    ] + [pltpu.SemaphoreType.DMA] * (2 * D),
)(_dot_body)


@jax.jit
def kernel(user_id, item_id, user_emb, user_bias, item_emb, item_bias, bias):
    uid = user_id.astype(jnp.int32)
    iid = item_id.astype(jnp.int32)
    bias16 = jnp.broadcast_to(bias.astype(jnp.float32), (L,))
    pb = _bias_call(uid, iid, user_bias, item_bias, bias16)
    return _dot_call(uid, iid, user_emb.T, item_emb.T, pb)


# speedup vs baseline: 5.3428x; 1.2184x over previous
import functools

import jax
import jax.numpy as jnp
from jax import lax
from jax.experimental import pallas as pl
from jax.experimental.pallas import tpu as pltpu
from jax.experimental.pallas import tpu_sc as plsc

L = 16
NC = 2
NS = 16
NW = NC * NS
B = 16384
K = 16
BPW = B // NW
D = 16
GENS = BPW // D
BLK = 128
W = BLK

_mesh = plsc.VectorSubcoreMesh(core_axis_name="c", subcore_axis_name="s")


def _bias_body(uid_hbm, iid_hbm, ubias_hbm, ibias_hbm, bias_hbm, out_hbm,
               uidx_v, iidx_v, ub_v, ib_v, bias_v, out_v, bsem):
    wid = lax.axis_index("s") * NC + lax.axis_index("c")
    base = wid * BPW
    pltpu.sync_copy(uid_hbm.at[pl.ds(base, BPW)], uidx_v)
    pltpu.sync_copy(iid_hbm.at[pl.ds(base, BPW)], iidx_v)
    bcp1 = pltpu.async_copy(ubias_hbm.at[uidx_v], ub_v, bsem)
    bcp2 = pltpu.async_copy(ibias_hbm.at[iidx_v], ib_v, bsem)
    pltpu.sync_copy(bias_hbm, bias_v)
    bcp1.wait()
    bcp2.wait()
    bvec = bias_v[...]

    def group(g, carry):
        rb = g * L
        out_v[pl.ds(rb, L)] = ub_v[pl.ds(rb, L)] + ib_v[pl.ds(rb, L)] + bvec
        return carry

    lax.fori_loop(0, BPW // L, group, 0)
    pltpu.sync_copy(out_v, out_hbm.at[pl.ds(base, BPW)])


_bias_call = functools.partial(
    pl.kernel,
    out_type=jax.ShapeDtypeStruct((B,), jnp.float32),
    mesh=_mesh,
    compiler_params=pltpu.CompilerParams(needs_layout_passes=False,
                                         use_tc_tiling_on_sc=False),
    scratch_types=[
        pltpu.VMEM((BPW,), jnp.int32),
        pltpu.VMEM((BPW,), jnp.int32),
        pltpu.VMEM((BPW,), jnp.float32),
        pltpu.VMEM((BPW,), jnp.float32),
        pltpu.VMEM((L,), jnp.float32),
        pltpu.VMEM((BPW,), jnp.float32),
        pltpu.SemaphoreType.DMA,
    ],
)(_bias_body)


def _dot_body(uid_hbm, iid_hbm, uembt_hbm, iembt_hbm, pb_hbm, out_hbm,
              uidx_v, iidx_v, uring_v, iring_v, pb_v, out_v, *slot_sems):
    usems = slot_sems
    isems = slot_sems
    wid = lax.axis_index("s") * NC + lax.axis_index("c")
    base = wid * BPW

    pltpu.sync_copy(uid_hbm.at[pl.ds(base, BPW)], uidx_v.at[pl.ds(0, BPW)])
    pltpu.sync_copy(iid_hbm.at[pl.ds(base, BPW)], iidx_v.at[pl.ds(0, BPW)])
    pltpu.sync_copy(pb_hbm.at[pl.ds(base, BPW)], pb_v)

    def fire_gen(gen):
        uids = uidx_v[pl.ds(gen * D, L)]
        iids = iidx_v[pl.ds(gen * D, L)]
        for d in range(D):
            u_off = pl.multiple_of((uids[d] >> 7) << 7, BLK)
            i_off = pl.multiple_of((iids[d] >> 7) << 7, BLK)
            pltpu.async_copy(uembt_hbm.at[:, pl.ds(u_off, W)],
                             uring_v.at[d], usems[d])
            pltpu.async_copy(iembt_hbm.at[:, pl.ds(i_off, W)],
                             iring_v.at[d], isems[d])

    fire_gen(0)

    lane = lax.iota(jnp.int32, L)
    zeros = jnp.zeros((L,), jnp.int32)

    def gen_step(g2, carry):
        rb = g2 * L
        uids = uidx_v[pl.ds(rb, L)]
        iids = iidx_v[pl.ds(rb, L)]
        acc = pb_v[pl.ds(rb, L)]
        for h in range(L // D):
            g = g2 * (L // D) + h
            for d in range(D):
                pltpu.make_async_copy(uembt_hbm.at[:, pl.ds(0, W)],
                                      uring_v.at[d], usems[d]).wait()
                pltpu.make_async_copy(iembt_hbm.at[:, pl.ds(0, W)],
                                      iring_v.at[d], isems[d]).wait()
                lane_idx = h * D + d
                ucol = (uids[lane_idx] & (W - 1)) + zeros
                icol = (iids[lane_idx] & (W - 1)) + zeros
                uvec = plsc.load_gather(uring_v.at[d], [lane, ucol])
                ivec = plsc.load_gather(iring_v.at[d], [lane, icol])
                s = jnp.sum(uvec * ivec)
                acc = jnp.where(lane == lane_idx, acc + s, acc)

            @pl.when(g < GENS - 1)
            def _refire():
                fire_gen(g + 1)

        out_v[pl.ds(rb, L)] = acc
        return carry

    lax.fori_loop(0, BPW // L, gen_step, 0)

    pltpu.sync_copy(out_v, out_hbm.at[pl.ds(base, BPW)])


_dot_call = functools.partial(
    pl.kernel,
    out_type=jax.ShapeDtypeStruct((B,), jnp.float32),
    mesh=_mesh,
    compiler_params=pltpu.CompilerParams(needs_layout_passes=False),
    scratch_types=[
        pltpu.VMEM((BPW + L,), jnp.int32),
        pltpu.VMEM((BPW + L,), jnp.int32),
        pltpu.VMEM((D, K, W), jnp.float32),
        pltpu.VMEM((D, K, W), jnp.float32),
        pltpu.VMEM((BPW,), jnp.float32),
        pltpu.VMEM((BPW,), jnp.float32),
    ] + [pltpu.SemaphoreType.DMA] * D,
)(_dot_body)


@jax.jit
def kernel(user_id, item_id, user_emb, user_bias, item_emb, item_bias, bias):
    uid = user_id.astype(jnp.int32)
    iid = item_id.astype(jnp.int32)
    bias16 = jnp.broadcast_to(bias.astype(jnp.float32), (L,))
    pb = _bias_call(uid, iid, user_bias, item_bias, bias16)
    return _dot_call(uid, iid, user_emb.T, item_emb.T, pb)


# speedup vs baseline: 5.3521x; 1.0017x over previous
import functools

import jax
import jax.numpy as jnp
from jax import lax
from jax.experimental import pallas as pl
from jax.experimental.pallas import tpu as pltpu
from jax.experimental.pallas import tpu_sc as plsc

L = 16
NC = 2
NS = 16
NW = NC * NS
B = 16384
K = 16
BPW = B // NW
SLOTS = 8
E = 2
GENS = BPW // (SLOTS * E)
BLK = 128

_mesh = plsc.VectorSubcoreMesh(core_axis_name="c", subcore_axis_name="s")


def _bias_body(uid_hbm, iid_hbm, ubias_hbm, ibias_hbm, bias_hbm, out_hbm,
               uidx_v, iidx_v, ub_v, ib_v, bias_v, out_v, bsem):
    wid = lax.axis_index("s") * NC + lax.axis_index("c")
    base = wid * BPW
    pltpu.sync_copy(uid_hbm.at[pl.ds(base, BPW)], uidx_v)
    pltpu.sync_copy(iid_hbm.at[pl.ds(base, BPW)], iidx_v)
    bcp1 = pltpu.async_copy(ubias_hbm.at[uidx_v], ub_v, bsem)
    bcp2 = pltpu.async_copy(ibias_hbm.at[iidx_v], ib_v, bsem)
    pltpu.sync_copy(bias_hbm, bias_v)
    bcp1.wait()
    bcp2.wait()
    bvec = bias_v[...]

    def group(g, carry):
        rb = g * L
        out_v[pl.ds(rb, L)] = ub_v[pl.ds(rb, L)] + ib_v[pl.ds(rb, L)] + bvec
        return carry

    lax.fori_loop(0, BPW // L, group, 0)
    pltpu.sync_copy(out_v, out_hbm.at[pl.ds(base, BPW)])


_bias_call = functools.partial(
    pl.kernel,
    out_type=jax.ShapeDtypeStruct((B,), jnp.float32),
    mesh=_mesh,
    compiler_params=pltpu.CompilerParams(needs_layout_passes=False,
                                         use_tc_tiling_on_sc=False),
    scratch_types=[
        pltpu.VMEM((BPW,), jnp.int32),
        pltpu.VMEM((BPW,), jnp.int32),
        pltpu.VMEM((BPW,), jnp.float32),
        pltpu.VMEM((BPW,), jnp.float32),
        pltpu.VMEM((L,), jnp.float32),
        pltpu.VMEM((BPW,), jnp.float32),
        pltpu.SemaphoreType.DMA,
    ],
)(_bias_body)


def _dot_body(uid_hbm, iid_hbm, uembt_hbm, iembt_hbm, pb_hbm, out_hbm,
              uidx_v, iidx_v, uring_v, iring_v, pb_v, out_v, *slot_sems):
    usems = slot_sems
    isems = slot_sems
    wid = lax.axis_index("s") * NC + lax.axis_index("c")
    base = wid * BPW

    pltpu.sync_copy(uid_hbm.at[pl.ds(base, BPW)], uidx_v.at[pl.ds(0, BPW)])
    pltpu.sync_copy(iid_hbm.at[pl.ds(base, BPW)], iidx_v.at[pl.ds(0, BPW)])
    pltpu.sync_copy(pb_hbm.at[pl.ds(base, BPW)], pb_v)

    def fire_gen(gen):
        uids = uidx_v[pl.ds(gen * L, L)]
        iids = iidx_v[pl.ds(gen * L, L)]
        for d in range(SLOTS):
            for e in range(E):
                li = d * E + e
                u_off = pl.multiple_of((uids[li] >> 7) << 7, BLK)
                i_off = pl.multiple_of((iids[li] >> 7) << 7, BLK)
                pltpu.async_copy(uembt_hbm.at[:, pl.ds(u_off, BLK)],
                                 uring_v.at[d, e], usems[d])
                pltpu.async_copy(iembt_hbm.at[:, pl.ds(i_off, BLK)],
                                 iring_v.at[d, e], isems[d])

    fire_gen(0)

    lane = lax.iota(jnp.int32, L)
    zeros = jnp.zeros((L,), jnp.int32)

    def gen_step(g, carry):
        rb = g * L
        uids = uidx_v[pl.ds(rb, L)]
        iids = iidx_v[pl.ds(rb, L)]
        acc = pb_v[pl.ds(rb, L)]
        for d in range(SLOTS):
            for e in range(E):
                pltpu.make_async_copy(uembt_hbm.at[:, pl.ds(0, BLK)],
                                      uring_v.at[d, e], usems[d]).wait()
                pltpu.make_async_copy(iembt_hbm.at[:, pl.ds(0, BLK)],
                                      iring_v.at[d, e], isems[d]).wait()
            for e in range(E):
                li = d * E + e
                ucol = (uids[li] & (BLK - 1)) + zeros
                icol = (iids[li] & (BLK - 1)) + zeros
                uvec = plsc.load_gather(uring_v.at[d, e], [lane, ucol])
                ivec = plsc.load_gather(iring_v.at[d, e], [lane, icol])
                s = jnp.sum(uvec * ivec)
                acc = jnp.where(lane == li, acc + s, acc)

        @pl.when(g < GENS - 1)
        def _refire():
            fire_gen(g + 1)

        out_v[pl.ds(rb, L)] = acc
        return carry

    lax.fori_loop(0, GENS, gen_step, 0)

    pltpu.sync_copy(out_v, out_hbm.at[pl.ds(base, BPW)])


_dot_call = functools.partial(
    pl.kernel,
    out_type=jax.ShapeDtypeStruct((B,), jnp.float32),
    mesh=_mesh,
    compiler_params=pltpu.CompilerParams(needs_layout_passes=False),
    scratch_types=[
        pltpu.VMEM((BPW + L,), jnp.int32),
        pltpu.VMEM((BPW + L,), jnp.int32),
        pltpu.VMEM((SLOTS, E, K, BLK), jnp.float32),
        pltpu.VMEM((SLOTS, E, K, BLK), jnp.float32),
        pltpu.VMEM((BPW,), jnp.float32),
        pltpu.VMEM((BPW,), jnp.float32),
    ] + [pltpu.SemaphoreType.DMA] * SLOTS,
)(_dot_body)


@jax.jit
def kernel(user_id, item_id, user_emb, user_bias, item_emb, item_bias, bias):
    uid = user_id.astype(jnp.int32)
    iid = item_id.astype(jnp.int32)
    bias16 = jnp.broadcast_to(bias.astype(jnp.float32), (L,))
    pb = _bias_call(uid, iid, user_bias, item_bias, bias16)
    return _dot_call(uid, iid, user_emb.T, item_emb.T, pb)


# speedup vs baseline: 5.5649x; 1.0398x over previous
import functools

import jax
import jax.numpy as jnp
from jax import lax
from jax.experimental import pallas as pl
from jax.experimental.pallas import tpu as pltpu
from jax.experimental.pallas import tpu_sc as plsc

L = 16
NC = 2
NS = 16
NW = NC * NS
B = 16384
K = 16
BPW = B // NW
SLOTS = 8
E = 2
GENS = BPW // (SLOTS * E)
BLK = 128

_mesh = plsc.VectorSubcoreMesh(core_axis_name="c", subcore_axis_name="s")


def _body(uid_hbm, iid_hbm, uembt_hbm, ubias_hbm, iembt_hbm, ibias_hbm,
          bias_hbm, out_hbm,
          uidx_v, iidx_v, uring_v, iring_v, ub_v, ib_v, bias_v, out_v,
          bsem, *slot_sems):
    wid = lax.axis_index("s") * NC + lax.axis_index("c")
    base = wid * BPW

    pltpu.sync_copy(uid_hbm.at[pl.ds(base, BPW)], uidx_v)
    pltpu.sync_copy(iid_hbm.at[pl.ds(base, BPW)], iidx_v)

    bcp1 = pltpu.async_copy(ubias_hbm.at[uidx_v], ub_v, bsem)
    bcp2 = pltpu.async_copy(ibias_hbm.at[iidx_v], ib_v, bsem)
    pltpu.sync_copy(bias_hbm, bias_v)

    def fire_gen(gen):
        uids = uidx_v[pl.ds(gen * L, L)]
        iids = iidx_v[pl.ds(gen * L, L)]
        for d in range(SLOTS):
            for e in range(E):
                li = d * E + e
                u_off = pl.multiple_of((uids[li] >> 7) << 7, BLK)
                i_off = pl.multiple_of((iids[li] >> 7) << 7, BLK)
                pltpu.async_copy(uembt_hbm.at[:, pl.ds(u_off, BLK)],
                                 uring_v.at[d, e], slot_sems[d])
                pltpu.async_copy(iembt_hbm.at[:, pl.ds(i_off, BLK)],
                                 iring_v.at[d, e], slot_sems[d])

    fire_gen(0)

    bcp1.wait()
    bcp2.wait()
    bvec = bias_v[...]
    lane = lax.iota(jnp.int32, L)
    zeros = jnp.zeros((L,), jnp.int32)

    def gen_step(g, carry):
        rb = g * L
        uids = uidx_v[pl.ds(rb, L)]
        iids = iidx_v[pl.ds(rb, L)]
        acc = ub_v[pl.ds(rb, L)] + ib_v[pl.ds(rb, L)] + bvec
        for d in range(SLOTS):
            for e in range(E):
                pltpu.make_async_copy(uembt_hbm.at[:, pl.ds(0, BLK)],
                                      uring_v.at[d, e], slot_sems[d]).wait()
                pltpu.make_async_copy(iembt_hbm.at[:, pl.ds(0, BLK)],
                                      iring_v.at[d, e], slot_sems[d]).wait()
            for e in range(E):
                li = d * E + e
                ucol = (uids[li] & (BLK - 1)) + zeros
                icol = (iids[li] & (BLK - 1)) + zeros
                uvec = plsc.load_gather(uring_v.at[d, e], [lane, ucol])
                ivec = plsc.load_gather(iring_v.at[d, e], [lane, icol])
                s = jnp.sum(uvec * ivec)
                acc = jnp.where(lane == li, acc + s, acc)

        @pl.when(g < GENS - 1)
        def _refire():
            fire_gen(g + 1)

        out_v[pl.ds(rb, L)] = acc
        return carry

    lax.fori_loop(0, GENS, gen_step, 0)

    pltpu.sync_copy(out_v, out_hbm.at[pl.ds(base, BPW)])


_sc_call = functools.partial(
    pl.kernel,
    out_type=jax.ShapeDtypeStruct((B,), jnp.float32),
    mesh=_mesh,
    compiler_params=pltpu.CompilerParams(needs_layout_passes=False),
    scratch_types=[
        pltpu.VMEM((BPW,), jnp.int32),
        pltpu.VMEM((BPW,), jnp.int32),
        pltpu.VMEM((SLOTS, E, K, BLK), jnp.float32),
        pltpu.VMEM((SLOTS, E, K, BLK), jnp.float32),
        pltpu.VMEM((BPW,), jnp.float32),
        pltpu.VMEM((BPW,), jnp.float32),
        pltpu.VMEM((L,), jnp.float32),
        pltpu.VMEM((BPW,), jnp.float32),
        pltpu.SemaphoreType.DMA,
    ] + [pltpu.SemaphoreType.DMA] * SLOTS,
)(_body)


@jax.jit
def kernel(user_id, item_id, user_emb, user_bias, item_emb, item_bias, bias):
    bias16 = jnp.broadcast_to(bias.astype(jnp.float32), (L,))
    return _sc_call(user_id.astype(jnp.int32), item_id.astype(jnp.int32),
                    user_emb.T, user_bias, item_emb.T, item_bias, bias16)
